# Initial kernel scaffold; baseline (speedup 1.0000x reference)
import jax
import jax.numpy as jnp
from jax.experimental import pallas as pl


def kernel(x, edges, a, W, attention_vectors):
    raise NotImplementedError("write your pallas kernel here")


# speedup vs baseline: 11.2959x; 11.2959x over previous
import functools

import jax
import jax.numpy as jnp
from jax import lax
from jax.experimental import pallas as pl
from jax.experimental.pallas import tpu as pltpu
from jax.experimental.pallas import tpu_sc as plsc

NEG_SLOPE = 0.2
NC = 2
NS = 16
NW = NC * NS
L = 16


def _prologue_body(x_ref, w_ref, av_ref, h_ref, us_ref, m_ref):
    x = x_ref[...]
    w = w_ref[...]
    av = av_ref[...]
    wu = lax.dot(av[:, 0:2], w, preferred_element_type=jnp.float32)
    wv = lax.dot(av[:, 2:4], w, preferred_element_type=jnp.float32)
    g = jnp.concatenate([w, wu, wv], axis=0)
    big = lax.dot_general(x, g, (((1,), (1,)), ((), ())),
                          preferred_element_type=jnp.float32)
    h_dim = w.shape[0]
    heads = av.shape[0]
    us = big[:, h_dim:h_dim + heads]
    vd = big[:, h_dim + heads:h_dim + 2 * heads]
    h_ref[...] = big[:, 0:h_dim]
    us_ref[...] = us
    bound = jnp.max(us, axis=0) + jnp.max(vd, axis=0)
    m = jnp.max(jnp.maximum(bound, NEG_SLOPE * bound))
    m_ref[...] = jnp.full((L,), m, dtype=jnp.float32)


def _prologue(x, w, av):
    n = x.shape[0]
    heads = av.shape[0]
    h_dim = w.shape[0]
    return pl.pallas_call(
        _prologue_body,
        out_shape=[
            jax.ShapeDtypeStruct((n, h_dim), jnp.float32),
            jax.ShapeDtypeStruct((n, heads), jnp.float32),
            jax.ShapeDtypeStruct((L,), jnp.float32),
        ],
    )(x, w, av)


def _sc_body(heads, epw, edges_ref, h_ref, us_ref, av_ref, m_ref, out_ref,
             e_v, h_v, us_v, av_v, m_v, o_v):
    cid = lax.axis_index("c")
    sid = lax.axis_index("s")
    wid = sid * NC + cid

    pltpu.sync_copy(edges_ref.at[pl.ds(wid * (2 * epw), 2 * epw)], e_v)
    pltpu.sync_copy(h_ref, h_v)
    pltpu.sync_copy(us_ref, us_v)
    pltpu.sync_copy(av_ref, av_v)
    pltpu.sync_copy(m_ref, m_v)

    lanes2 = lax.iota(jnp.int32, L) * 2
    m_vec = m_v[...]
    av2 = [plsc.load_gather(av_v, [jnp.full((L,), 4 * h + 2, jnp.int32)])
           for h in range(heads)]
    av3 = [plsc.load_gather(av_v, [jnp.full((L,), 4 * h + 3, jnp.int32)])
           for h in range(heads)]
    colh = [jnp.full((L,), h, jnp.int32) for h in range(heads)]

    def body(i, accs):
        ebase = lanes2 + i * (2 * L)
        s = plsc.load_gather(e_v, [ebase])
        d = plsc.load_gather(e_v, [ebase + 1])
        d2 = d * 2
        s8 = s * heads
        hd0 = plsc.load_gather(h_v, [d2])
        hd1 = plsc.load_gather(h_v, [d2 + 1])
        new = []
        for h in range(heads):
            us_h = plsc.load_gather(us_v, [s8 + colh[h]])
            z = us_h + av2[h] * hd0 + av3[h] * hd1
            y = jnp.maximum(z, NEG_SLOPE * z)
            p = jnp.exp(y - m_vec)
            new.append(accs[3 * h] + p)
            new.append(accs[3 * h + 1] + p * hd0)
            new.append(accs[3 * h + 2] + p * hd1)
        return tuple(new)

    init = tuple(jnp.zeros((L,), jnp.float32) for _ in range(3 * heads))
    accs = lax.fori_loop(0, epw // L, body, init)

    for h in range(heads):
        o_v[pl.ds((0 * heads + h) * L, L)] = accs[3 * h]
        o_v[pl.ds((1 * heads + h) * L, L)] = accs[3 * h + 1]
        o_v[pl.ds((2 * heads + h) * L, L)] = accs[3 * h + 2]
    nacc = 3 * heads * L
    pltpu.sync_copy(o_v, out_ref.at[pl.ds(wid * nacc, nacc)])


def _sc_main(edges_flat, h_flat, us_flat, av_flat, m):
    two_e = edges_flat.shape[0]
    epw = two_e // (2 * NW)
    heads = av_flat.shape[0] // 4
    mesh = plsc.VectorSubcoreMesh(core_axis_name="c", subcore_axis_name="s",
                                  num_cores=NC, num_subcores=NS)
    nacc = 3 * heads * L
    f = pl.kernel(
        functools.partial(_sc_body, heads, epw),
        out_type=jax.ShapeDtypeStruct((NW * nacc,), jnp.float32),
        mesh=mesh,
        compiler_params=pltpu.CompilerParams(needs_layout_passes=False),
        scratch_types=[
            pltpu.VMEM((2 * epw,), jnp.int32),
            pltpu.VMEM((h_flat.shape[0],), jnp.float32),
            pltpu.VMEM((us_flat.shape[0],), jnp.float32),
            pltpu.VMEM((av_flat.shape[0],), jnp.float32),
            pltpu.VMEM((L,), jnp.float32),
            pltpu.VMEM((nacc,), jnp.float32),
        ],
    )
    return f(edges_flat, h_flat, us_flat, av_flat, m)


def _epilogue_body(heads, p_ref, o_ref):
    t = jnp.sum(jnp.sum(p_ref[...], axis=0), axis=-1)
    s = t[0:heads]
    t0 = t[heads:2 * heads]
    t1 = t[2 * heads:3 * heads]
    o_ref[...] = jax.nn.sigmoid(jnp.stack([t0 / s, t1 / s], axis=0))


def _epilogue(partials, heads):
    return pl.pallas_call(
        functools.partial(_epilogue_body, heads),
        out_shape=jax.ShapeDtypeStruct((2, heads), jnp.float32),
    )(partials)


def kernel(x, edges, a, W, attention_vectors):
    del a
    e = edges.shape[0]
    heads = attention_vectors.shape[0]
    assert e % (NW * L) == 0

    h2, us2, m = _prologue(x, W, attention_vectors)
    partials = _sc_main(
        edges.astype(jnp.int32).reshape(-1),
        h2.reshape(-1),
        us2.reshape(-1),
        attention_vectors.reshape(-1),
        m,
    )
    out28 = _epilogue(partials.reshape(NW, 3 * heads, L), heads)
    return out28.T


# speedup vs baseline: 32.3572x; 2.8645x over previous
import functools

import jax
import jax.numpy as jnp
from jax import lax
from jax.experimental import pallas as pl
from jax.experimental.pallas import tpu as pltpu
from jax.experimental.pallas import tpu_sc as plsc

NEG_SLOPE = 0.2
NC = 2
NS = 16
NW = NC * NS
L = 16


def _prologue_body(x_ref, w_ref, av_ref, h_ref, us_ref, m_ref):
    x = x_ref[...]
    w = w_ref[...]
    av = av_ref[...]
    wu = lax.dot(av[:, 0:2], w, preferred_element_type=jnp.float32)
    wv = lax.dot(av[:, 2:4], w, preferred_element_type=jnp.float32)
    g = jnp.concatenate([w, wu, wv], axis=0)
    big = lax.dot_general(x, g, (((1,), (1,)), ((), ())),
                          preferred_element_type=jnp.float32)
    h_dim = w.shape[0]
    heads = av.shape[0]
    us = big[:, h_dim:h_dim + heads]
    vd = big[:, h_dim + heads:h_dim + 2 * heads]
    h_ref[...] = big[:, 0:h_dim]
    us_ref[...] = us
    bound = jnp.max(us, axis=0) + jnp.max(vd, axis=0)
    m = jnp.max(jnp.maximum(bound, NEG_SLOPE * bound))
    m_ref[...] = jnp.full((L,), m, dtype=jnp.float32)


def _prologue(x, w, av):
    n = x.shape[0]
    heads = av.shape[0]
    h_dim = w.shape[0]
    return pl.pallas_call(
        _prologue_body,
        out_shape=[
            jax.ShapeDtypeStruct((n, h_dim), jnp.float32),
            jax.ShapeDtypeStruct((n, heads), jnp.float32),
            jax.ShapeDtypeStruct((L,), jnp.float32),
        ],
    )(x, w, av)


def _sc_body(heads, epw, es_ref, ed_ref, h_ref, us_ref, av_ref, m_ref, out_ref,
             es_v, ed_v, h_v, us_v, av_v, m_v, o_v):
    cid = lax.axis_index("c")
    sid = lax.axis_index("s")
    wid = sid * NC + cid

    pltpu.sync_copy(es_ref.at[pl.ds(wid * epw, epw)], es_v)
    pltpu.sync_copy(ed_ref.at[pl.ds(wid * epw, epw)], ed_v)
    pltpu.sync_copy(h_ref, h_v)
    pltpu.sync_copy(us_ref, us_v)
    pltpu.sync_copy(av_ref, av_v)
    pltpu.sync_copy(m_ref, m_v)

    m_vec = m_v[...]
    av2 = [plsc.load_gather(av_v, [jnp.full((L,), 4 * h + 2, jnp.int32)])
           for h in range(heads)]
    av3 = [plsc.load_gather(av_v, [jnp.full((L,), 4 * h + 3, jnp.int32)])
           for h in range(heads)]
    colh = [jnp.full((L,), h, jnp.int32) for h in range(heads)]

    def body(i, accs):
        s = es_v[pl.ds(i * L, L)]
        d = ed_v[pl.ds(i * L, L)]
        d2 = d * 2
        s8 = s * heads
        hd0 = plsc.load_gather(h_v, [d2])
        hd1 = plsc.load_gather(h_v, [d2 + 1])
        new = []
        for h in range(heads):
            us_h = plsc.load_gather(us_v, [s8 + colh[h]])
            z = us_h + av2[h] * hd0 + av3[h] * hd1
            y = jnp.maximum(z, NEG_SLOPE * z)
            p = jnp.exp(y - m_vec)
            new.append(accs[3 * h] + p)
            new.append(accs[3 * h + 1] + p * hd0)
            new.append(accs[3 * h + 2] + p * hd1)
        return tuple(new)

    init = tuple(jnp.zeros((L,), jnp.float32) for _ in range(3 * heads))
    accs = lax.fori_loop(0, epw // L, body, init)

    for h in range(heads):
        o_v[pl.ds((0 * heads + h) * L, L)] = accs[3 * h]
        o_v[pl.ds((1 * heads + h) * L, L)] = accs[3 * h + 1]
        o_v[pl.ds((2 * heads + h) * L, L)] = accs[3 * h + 2]
    nacc = 3 * heads * L
    pltpu.sync_copy(o_v, out_ref.at[pl.ds(wid * nacc, nacc)])


def _sc_main(es, ed, h_flat, us_flat, av_flat, m):
    epw = es.shape[0] // NW
    heads = av_flat.shape[0] // 4
    mesh = plsc.VectorSubcoreMesh(core_axis_name="c", subcore_axis_name="s",
                                  num_cores=NC, num_subcores=NS)
    nacc = 3 * heads * L
    f = pl.kernel(
        functools.partial(_sc_body, heads, epw),
        out_type=jax.ShapeDtypeStruct((NW * nacc,), jnp.float32),
        mesh=mesh,
        compiler_params=pltpu.CompilerParams(needs_layout_passes=False),
        scratch_types=[
            pltpu.VMEM((epw,), jnp.int32),
            pltpu.VMEM((epw,), jnp.int32),
            pltpu.VMEM((h_flat.shape[0],), jnp.float32),
            pltpu.VMEM((us_flat.shape[0],), jnp.float32),
            pltpu.VMEM((av_flat.shape[0],), jnp.float32),
            pltpu.VMEM((L,), jnp.float32),
            pltpu.VMEM((nacc,), jnp.float32),
        ],
    )
    return f(es, ed, h_flat, us_flat, av_flat, m)


def _epilogue_body(heads, p_ref, o_ref):
    t = jnp.sum(jnp.sum(p_ref[...], axis=0), axis=-1)
    s = t[0:heads]
    t0 = t[heads:2 * heads]
    t1 = t[2 * heads:3 * heads]
    o_ref[...] = jax.nn.sigmoid(jnp.stack([t0 / s, t1 / s], axis=0))


def _epilogue(partials, heads):
    return pl.pallas_call(
        functools.partial(_epilogue_body, heads),
        out_shape=jax.ShapeDtypeStruct((2, heads), jnp.float32),
    )(partials)


def kernel(x, edges, a, W, attention_vectors):
    del a
    e = edges.shape[0]
    heads = attention_vectors.shape[0]
    assert e % (NW * L) == 0

    h2, us2, m = _prologue(x, W, attention_vectors)
    e32 = edges.astype(jnp.int32)
    partials = _sc_main(
        e32[:, 0],
        e32[:, 1],
        h2.reshape(-1),
        us2.reshape(-1),
        attention_vectors.reshape(-1),
        m,
    )
    out28 = _epilogue(partials.reshape(NW, 3 * heads, L), heads)
    return out28.T
